# baseline (speedup 1.0000x reference)
import functools

import jax
import jax.numpy as jnp
from jax import lax
from jax.experimental import pallas as pl
from jax.experimental.pallas import tpu as pltpu
from jax.experimental.pallas import tpu_sc as plsc

D = 128
NC, NS = 2, 16
NW = NC * NS
CHUNK = 320


def _sc_gather(table, u_idx, v_idx, s_pad):
    per_w = s_pad // NW
    n_chunk = per_w // CHUNK
    mesh = plsc.VectorSubcoreMesh(core_axis_name="c", subcore_axis_name="s")

    @functools.partial(
        pl.kernel,
        out_type=(
            jax.ShapeDtypeStruct((s_pad, D), jnp.float32),
            jax.ShapeDtypeStruct((s_pad, D), jnp.float32),
        ),
        mesh=mesh,
        scratch_types=[
            pltpu.VMEM((CHUNK,), jnp.int32),
            pltpu.VMEM((CHUNK,), jnp.int32),
            pltpu.VMEM((CHUNK, D), jnp.float32),
            pltpu.VMEM((CHUNK, D), jnp.float32),
            pltpu.SemaphoreType.DMA,
            pltpu.SemaphoreType.DMA,
        ],
    )
    def gather_k(table_h, u_h, v_h, out_u, out_v, idxu, idxv, rows_u, rows_v,
                 semu, semv):
        wid = lax.axis_index("s") * NC + lax.axis_index("c")
        base = wid * per_w

        def body(ci, carry):
            off = base + ci * CHUNK
            pltpu.sync_copy(u_h.at[pl.ds(off, CHUNK)], idxu)
            pltpu.sync_copy(v_h.at[pl.ds(off, CHUNK)], idxv)
            cu = pltpu.async_copy(table_h.at[idxu], rows_u, semu)
            cv = pltpu.async_copy(table_h.at[idxv], rows_v, semv)
            cu.wait()
            cv.wait()
            pltpu.sync_copy(rows_u, out_u.at[pl.ds(off, CHUNK)])
            pltpu.sync_copy(rows_v, out_v.at[pl.ds(off, CHUNK)])
            return carry

        lax.fori_loop(0, n_chunk, body, 0)

    return gather_k(table, u_idx, v_idx)


def _tc_head(zu, zv, wl1, bl1, wl2, bl2, block):
    s_pad = zu.shape[0]
    grid = s_pad // block

    def head_k(zu_ref, zv_ref, w1_ref, b1_ref, w2_ref, b2_ref, out_ref):
        z = zu_ref[...] * zv_ref[...]
        h = jnp.dot(z, w1_ref[...], preferred_element_type=jnp.float32)
        h = jnp.maximum(h + b1_ref[...], 0.0)
        o = jnp.dot(h, w2_ref[...], preferred_element_type=jnp.float32)
        out_ref[...] = o + b2_ref[...]

    return pl.pallas_call(
        head_k,
        grid=(grid,),
        in_specs=[
            pl.BlockSpec((block, D), lambda i: (i, 0)),
            pl.BlockSpec((block, D), lambda i: (i, 0)),
            pl.BlockSpec((D, D), lambda i: (0, 0)),
            pl.BlockSpec((1, D), lambda i: (0, 0)),
            pl.BlockSpec((D, 2), lambda i: (0, 0)),
            pl.BlockSpec((1, 2), lambda i: (0, 0)),
        ],
        out_specs=pl.BlockSpec((block, 2), lambda i: (i, 0)),
        out_shape=jax.ShapeDtypeStruct((s_pad, 2), jnp.float32),
    )(zu, zv, wl1, bl1, wl2, bl2)


def kernel(x_feature, edge_index, samples, edges, W1, b1, W2, b2,
           Wl1, bl1, Wl2, bl2):
    s = samples.shape[0]
    step = NW * CHUNK
    s_pad = ((s + step - 1) // step) * step
    uv = jnp.zeros((2, s_pad), jnp.int32).at[:, :s].set(samples.T)
    zu, zv = _sc_gather(x_feature, uv[0], uv[1], s_pad)
    out = _tc_head(zu, zv, Wl1, bl1.reshape(1, D), Wl2, bl2.reshape(1, 2),
                   block=1024)
    return out[:s]


# speedup vs baseline: 1.1446x; 1.1446x over previous
import functools

import jax
import jax.numpy as jnp
from jax import lax
from jax.experimental import pallas as pl
from jax.experimental.pallas import tpu as pltpu
from jax.experimental.pallas import tpu_sc as plsc

D = 128
L = 16
NC, NS = 2, 16
NW = NC * NS
CHUNK = 200
NBUF = 2


def _sc_gather_mul(table, u_idx, v_idx, s_pad):
    per_w = s_pad // NW
    n_chunk = per_w // CHUNK
    mesh = plsc.VectorSubcoreMesh(core_axis_name="c", subcore_axis_name="s")

    @functools.partial(
        pl.kernel,
        out_type=jax.ShapeDtypeStruct((s_pad, D), jnp.float32),
        mesh=mesh,
        scratch_types=[
            pltpu.VMEM((per_w,), jnp.int32),
            pltpu.VMEM((per_w,), jnp.int32),
            pltpu.VMEM((NBUF, CHUNK, D), jnp.float32),
            pltpu.VMEM((NBUF, CHUNK, D), jnp.float32),
            pltpu.SemaphoreType.DMA((NBUF,)),
            pltpu.SemaphoreType.DMA((NBUF,)),
        ],
    )
    def gather_k(table_h, u_h, v_h, out_h, u_all, v_all, rows_u, rows_v,
                 semu, semv):
        wid = lax.axis_index("s") * NC + lax.axis_index("c")
        base = wid * per_w
        pltpu.sync_copy(u_h.at[pl.ds(base, per_w)], u_all)
        pltpu.sync_copy(v_h.at[pl.ds(base, per_w)], v_all)

        def fire(ci, b):
            cu = pltpu.async_copy(table_h.at[u_all.at[pl.ds(ci * CHUNK, CHUNK)]],
                                  rows_u.at[b], semu.at[b])
            cv = pltpu.async_copy(table_h.at[v_all.at[pl.ds(ci * CHUNK, CHUNK)]],
                                  rows_v.at[b], semv.at[b])
            return cu, cv

        def drain(b):
            pltpu.make_async_copy(table_h.at[u_all.at[pl.ds(0, CHUNK)]],
                                  rows_u.at[b], semu.at[b]).wait()
            pltpu.make_async_copy(table_h.at[v_all.at[pl.ds(0, CHUNK)]],
                                  rows_v.at[b], semv.at[b]).wait()

        for b in range(min(NBUF, n_chunk)):
            fire(b, b)
        for ci in range(n_chunk):
            b = ci % NBUF
            drain(b)

            def mul_row(i, carry):
                for j in range(D // L):
                    sl = pl.ds(j * L, L)
                    rows_u[b, i, sl] = rows_u[b, i, sl] * rows_v[b, i, sl]
                return carry

            lax.fori_loop(0, CHUNK, mul_row, 0)
            pltpu.sync_copy(rows_u.at[b],
                            out_h.at[pl.ds(base + ci * CHUNK, CHUNK)])
            if ci + NBUF < n_chunk:
                fire(ci + NBUF, b)

    return gather_k(table, u_idx, v_idx)


def _tc_head(z, wl1, bl1, wl2, bl2, block):
    s_pad = z.shape[0]
    grid = s_pad // block

    def head_k(z_ref, w1_ref, b1_ref, w2_ref, b2_ref, out_ref):
        h = jnp.dot(z_ref[...], w1_ref[...], preferred_element_type=jnp.float32)
        h = jnp.maximum(h + b1_ref[...], 0.0)
        o = jnp.dot(h, w2_ref[...], preferred_element_type=jnp.float32)
        out_ref[...] = o + b2_ref[...]

    return pl.pallas_call(
        head_k,
        grid=(grid,),
        in_specs=[
            pl.BlockSpec((block, D), lambda i: (i, 0)),
            pl.BlockSpec((D, D), lambda i: (0, 0)),
            pl.BlockSpec((1, D), lambda i: (0, 0)),
            pl.BlockSpec((D, 2), lambda i: (0, 0)),
            pl.BlockSpec((1, 2), lambda i: (0, 0)),
        ],
        out_specs=pl.BlockSpec((block, 2), lambda i: (i, 0)),
        out_shape=jax.ShapeDtypeStruct((s_pad, 2), jnp.float32),
    )(z, wl1, bl1, wl2, bl2)


def kernel(x_feature, edge_index, samples, edges, W1, b1, W2, b2,
           Wl1, bl1, Wl2, bl2):
    s = samples.shape[0]
    step = NW * CHUNK
    s_pad = ((s + step - 1) // step) * step
    uv = jnp.zeros((2, s_pad), jnp.int32).at[:, :s].set(samples.T)
    z = _sc_gather_mul(x_feature, uv[0], uv[1], s_pad)
    out = _tc_head(z, Wl1, bl1.reshape(1, D), Wl2, bl2.reshape(1, 2),
                   block=1280)
    return out[:s]
